# Initial kernel scaffold; baseline (speedup 1.0000x reference)
import jax
import jax.numpy as jnp
from jax.experimental import pallas as pl


def kernel(x, edge_index, W1, b1, g1w, g1b, W2, b2, g2w, g2b, W3, b3):
    raise NotImplementedError("write your pallas kernel here")


# speedup vs baseline: 14.8426x; 14.8426x over previous
import functools

import jax
import jax.numpy as jnp
from jax import lax
from jax.experimental import pallas as pl
from jax.experimental.pallas import tpu as pltpu
from jax.experimental.pallas import tpu_sc as plsc

N = 10000
E = 160000
NCORE = 2
NSUB = 16
EPT = E // NSUB
K = 80
NCH = EPT // K
RSTEP = 624
RCOPY = 640
EPS = 1e-5
BR = 1000
GRID = N // BR


@functools.lru_cache(maxsize=None)
def _sc_gather_scatter(dc):
    mesh = plsc.VectorSubcoreMesh(core_axis_name="c", subcore_axis_name="s")

    @functools.partial(
        pl.kernel,
        out_type=jax.ShapeDtypeStruct((NCORE, N, dc), jnp.float32),
        mesh=mesh,
        scratch_types=[
            pltpu.VMEM((NCH, K), jnp.int32),
            pltpu.VMEM((NCH, K), jnp.int32),
            pltpu.VMEM((K, dc), jnp.float32),
            pltpu.VMEM((K, dc), jnp.float32),
            pltpu.VMEM_SHARED((N, dc), jnp.float32),
            pltpu.SemaphoreType.DMA,
            pltpu.SemaphoreType.DMA,
        ],
        compiler_params=pltpu.CompilerParams(use_tc_tiling_on_sc=False),
    )
    def k(table, srcoff, dstr, out, src_v, dst_v, g0, g1, acc, sem0, sem1):
        c = lax.axis_index("c")
        s = lax.axis_index("s")
        base = s * RSTEP
        pltpu.sync_copy(srcoff.at[c, s], src_v)
        pltpu.sync_copy(dstr.at[s], dst_v)
        pltpu.sync_copy(table.at[pl.ds(c * N + base, RCOPY)],
                        acc.at[pl.ds(base, RCOPY)])
        plsc.subcore_barrier()
        pltpu.async_copy(table.at[src_v.at[0]], g0, sem0)

        @pl.loop(0, NCH - 1, step=2)
        def _(j):
            pltpu.async_copy(table.at[src_v.at[j + 1]], g1, sem1)
            pltpu.make_async_copy(table.at[src_v.at[j]], g0, sem0).wait()
            pltpu.sync_copy(g0, acc.at[dst_v.at[j]], add=True)
            pltpu.async_copy(table.at[src_v.at[j + 2]], g0, sem0)
            pltpu.make_async_copy(table.at[src_v.at[j + 1]], g1, sem1).wait()
            pltpu.sync_copy(g1, acc.at[dst_v.at[j + 1]], add=True)

        pltpu.make_async_copy(table.at[src_v.at[NCH - 1]], g0, sem0).wait()
        pltpu.sync_copy(g0, acc.at[dst_v.at[NCH - 1]], add=True)
        plsc.subcore_barrier()
        pltpu.sync_copy(acc.at[pl.ds(base, RCOPY)],
                        out.at[c, pl.ds(base, RCOPY)])

    return k


def _group_mats(half_dim, group):
    ngrp = half_dim // group
    ri = lax.broadcasted_iota(jnp.int32, (half_dim, ngrp), 0) // group
    ci = lax.broadcasted_iota(jnp.int32, (half_dim, ngrp), 1)
    m = jnp.where(ri == ci, 1.0, 0.0).astype(jnp.float32)
    rit = lax.broadcasted_iota(jnp.int32, (ngrp, half_dim), 0)
    cit = lax.broadcasted_iota(jnp.int32, (ngrp, half_dim), 1) // group
    mt = jnp.where(rit == cit, 1.0, 0.0).astype(jnp.float32)
    return m, mt


def _tc_pre_body(x_ref, w_ref, d_ref, o_ref):
    dis = lax.rsqrt(d_ref[:, 0:1])
    h = jnp.dot(x_ref[...], w_ref[...], preferred_element_type=jnp.float32)
    hp = h * dis
    o_ref[0] = hp[:, :128]
    o_ref[1] = hp[:, 128:]


def _tc_pre(x, w1, degh):
    return pl.pallas_call(
        _tc_pre_body,
        grid=(GRID,),
        in_specs=[
            pl.BlockSpec((BR, 256), lambda i: (i, 0)),
            pl.BlockSpec((256, 256), lambda i: (0, 0)),
            pl.BlockSpec((BR, 16), lambda i: (i, 0)),
        ],
        out_specs=pl.BlockSpec((2, BR, 128), lambda i: (0, i, 0)),
        out_shape=jax.ShapeDtypeStruct((2, N, 128), jnp.float32),
    )(x, w1, degh)


def _tc_mid_body(dn, a_ref, d_ref, b_ref, gw_ref, gb_ref, w_ref, o_ref):
    dis = lax.rsqrt(d_ref[:, 0:1])
    m, mt = _group_mats(128, 32)
    ys = []
    for half in range(2):
        lo, hi = half * 128, half * 128 + 128
        u = a_ref[half] * dis + b_ref[:, lo:hi]
        s4 = jnp.dot(u, m, preferred_element_type=jnp.float32) * (1.0 / 32.0)
        mean = jnp.dot(s4, mt, preferred_element_type=jnp.float32)
        q4 = jnp.dot(u * u, m, preferred_element_type=jnp.float32) * (1.0 / 32.0)
        q = jnp.dot(q4, mt, preferred_element_type=jnp.float32)
        var = q - mean * mean
        y = (u - mean) * lax.rsqrt(var + EPS)
        y = y * gw_ref[:, lo:hi] + gb_ref[:, lo:hi]
        ys.append(jnp.maximum(y, 0.0))
    h = (jnp.dot(ys[0], w_ref[:128, :], preferred_element_type=jnp.float32)
         + jnp.dot(ys[1], w_ref[128:, :], preferred_element_type=jnp.float32))
    hp = h * dis
    hd = dn // 2
    o_ref[0] = hp[:, :hd]
    o_ref[1] = hp[:, hd:]


def _tc_mid(acc, degh, b, gw, gb, w, dn):
    return pl.pallas_call(
        functools.partial(_tc_mid_body, dn),
        grid=(GRID,),
        in_specs=[
            pl.BlockSpec((2, BR, 128), lambda i: (0, i, 0)),
            pl.BlockSpec((BR, 16), lambda i: (i, 0)),
            pl.BlockSpec((1, 256), lambda i: (0, 0)),
            pl.BlockSpec((1, 256), lambda i: (0, 0)),
            pl.BlockSpec((1, 256), lambda i: (0, 0)),
            pl.BlockSpec((256, dn), lambda i: (0, 0)),
        ],
        out_specs=pl.BlockSpec((2, BR, dn // 2), lambda i: (0, i, 0)),
        out_shape=jax.ShapeDtypeStruct((2, N, dn // 2), jnp.float32),
    )(acc, degh, b, gw, gb, w)


def _tc_final_body(a_ref, d_ref, b_ref, o_ref):
    dis = lax.rsqrt(d_ref[:, 0:1])
    u = jnp.concatenate([a_ref[0], a_ref[1]], axis=1)
    z = u * dis + b_ref[:, :]
    col = lax.broadcasted_iota(jnp.int32, (BR, 64), 1)
    mask = col < 40
    zm = jnp.where(mask, z, -jnp.inf)
    mx = jnp.max(zm, axis=1, keepdims=True)
    ez = jnp.where(mask, jnp.exp(z - mx), 0.0)
    se = jnp.sum(ez, axis=1, keepdims=True)
    ls = z - mx - jnp.log(se)
    o_ref[...] = ls[:, :40]


def _tc_final(acc3, degh, b3p):
    return pl.pallas_call(
        _tc_final_body,
        grid=(GRID,),
        in_specs=[
            pl.BlockSpec((2, BR, 32), lambda i: (0, i, 0)),
            pl.BlockSpec((BR, 16), lambda i: (i, 0)),
            pl.BlockSpec((1, 64), lambda i: (0, 0)),
        ],
        out_specs=pl.BlockSpec((BR, 40), lambda i: (i, 0)),
        out_shape=jax.ShapeDtypeStruct((N, 40), jnp.float32),
    )(acc3, degh, b3p)


def kernel(x, edge_index, W1, b1, g1w, g1b, W2, b2, g2w, g2b, W3, b3):
    src_r = edge_index[0].reshape(NSUB, NCH, K)
    src_off = jnp.stack([src_r, src_r + N])
    dst_r = edge_index[1].reshape(NSUB, NCH, K)

    ones_tab = jnp.ones((2 * N, 16), jnp.float32)
    degh = _sc_gather_scatter(16)(ones_tab, src_off, dst_r)[0]

    hp1 = _tc_pre(x, W1, degh)
    acc1 = _sc_gather_scatter(128)(hp1.reshape(2 * N, 128), src_off, dst_r)
    hp2 = _tc_mid(acc1, degh, b1.reshape(1, 256), g1w.reshape(1, 256),
                  g1b.reshape(1, 256), W2, 256)
    acc2 = _sc_gather_scatter(128)(hp2.reshape(2 * N, 128), src_off, dst_r)
    w3p = jnp.pad(W3, ((0, 0), (0, 24)))
    hp3 = _tc_mid(acc2, degh, b2.reshape(1, 256), g2w.reshape(1, 256),
                  g2b.reshape(1, 256), w3p, 64)
    acc3 = _sc_gather_scatter(32)(hp3.reshape(2 * N, 32), src_off, dst_r)
    b3p = jnp.pad(b3, (0, 24)).reshape(1, 64)
    return _tc_final(acc3, degh, b3p)
